# Initial kernel scaffold; baseline (speedup 1.0000x reference)
import jax
import jax.numpy as jnp
from jax.experimental import pallas as pl


def kernel(x, edge_index, edge_attr, u, batch, W1, b1, W2, b2):
    raise NotImplementedError("write your pallas kernel here")


# speedup vs baseline: 3.2642x; 3.2642x over previous
import functools

import jax
import jax.numpy as jnp
from jax import lax
from jax.experimental import pallas as pl
from jax.experimental.pallas import tpu as pltpu
from jax.experimental.pallas import tpu_sc as plsc

N_NODES = 10000
N_EDGES = 320000
D_FEAT = 128
D_HID = 16

NUM_CORES = 2
NUM_SUBCORES = 16
NW = NUM_CORES * NUM_SUBCORES
EDGES_PER_W = N_EDGES // NW
CHUNK = 80
STEPS = EDGES_PER_W // CHUNK
ZROWS = 1000
SLAB = 624
TAIL = N_NODES - NUM_SUBCORES * SLAB


def _y_body(x_ref, w_ref, b_ref, o_ref):
    o_ref[...] = (
        jnp.dot(x_ref[...], w_ref[...], preferred_element_type=jnp.float32)
        + b_ref[...]
    )


def _e_body(ea_ref, w_ref, o_ref):
    o_ref[...] = jnp.dot(ea_ref[...], w_ref[...],
                         preferred_element_type=jnp.float32)


def _fin_body(x_ref, s_ref, c_ref, wa_ref, wb_ref, b_ref, o_ref):
    sums = s_ref[0] + s_ref[1]
    cnts = jnp.maximum(c_ref[0] + c_ref[1], 1.0)
    agg = sums / cnts
    o_ref[...] = jnp.maximum(
        jnp.dot(x_ref[...], wa_ref[...], preferred_element_type=jnp.float32)
        + jnp.dot(agg, wb_ref[...], preferred_element_type=jnp.float32)
        + b_ref[...],
        0.0,
    )


def _sc_body(row_ref, col_ref, y_ref, e_ref, sums_out, cnts_out,
             ridx, cidx, rows, ev, ones, zv, sums_sh, cnts_sh, sem):
    c = lax.axis_index("c")
    s = lax.axis_index("s")
    wid = s * NUM_CORES + c

    def _fill(i, _):
        ones[i] = jnp.ones((16,), jnp.float32)
        return _
    lax.fori_loop(0, CHUNK, _fill, None)

    def _zfill(i, _):
        zv[i] = jnp.zeros((16,), jnp.float32)
        return _
    lax.fori_loop(0, ZROWS, _zfill, None)

    @pl.when(s == 0)
    def _zero_spmem():
        def _z(k, _):
            sl = pl.ds(k * ZROWS, ZROWS)
            pltpu.sync_copy(zv, sums_sh.at[sl])
            pltpu.sync_copy(zv, cnts_sh.at[sl])
            return _
        lax.fori_loop(0, N_NODES // ZROWS, _z, None)

    plsc.subcore_barrier()

    def _step(j, _):
        base = wid * EDGES_PER_W + j * CHUNK
        pltpu.sync_copy(row_ref.at[pl.ds(base, CHUNK)], ridx)
        pltpu.sync_copy(col_ref.at[pl.ds(base, CHUNK)], cidx)
        pltpu.async_copy(y_ref.at[ridx], rows, sem).wait()
        pltpu.sync_copy(e_ref.at[pl.ds(base, CHUNK)], ev)

        def _cmp(i, _2):
            rows[i] = jnp.maximum(rows[i] + ev[i], 0.0)
            return _2
        lax.fori_loop(0, CHUNK, _cmp, None)

        pltpu.sync_copy(rows, sums_sh.at[cidx], add=True)
        pltpu.sync_copy(ones, cnts_sh.at[cidx], add=True)
        return _
    lax.fori_loop(0, STEPS, _step, None)

    plsc.subcore_barrier()

    sl = pl.ds(s * SLAB, SLAB)
    pltpu.sync_copy(sums_sh.at[sl], sums_out.at[c, sl])
    pltpu.sync_copy(cnts_sh.at[sl], cnts_out.at[c, sl])

    @pl.when(s == 0)
    def _tail():
        tl = pl.ds(NUM_SUBCORES * SLAB, TAIL)
        pltpu.sync_copy(sums_sh.at[tl], sums_out.at[c, tl])
        pltpu.sync_copy(cnts_sh.at[tl], cnts_out.at[c, tl])


@functools.partial(
    pl.kernel,
    mesh=plsc.VectorSubcoreMesh(core_axis_name="c", subcore_axis_name="s"),
    compiler_params=pltpu.CompilerParams(use_tc_tiling_on_sc=False),
    out_type=[
        jax.ShapeDtypeStruct((NUM_CORES, N_NODES, D_HID), jnp.float32),
        jax.ShapeDtypeStruct((NUM_CORES, N_NODES, D_HID), jnp.float32),
    ],
    scratch_types=[
        pltpu.VMEM((CHUNK,), jnp.int32),
        pltpu.VMEM((CHUNK,), jnp.int32),
        pltpu.VMEM((CHUNK, D_HID), jnp.float32),
        pltpu.VMEM((CHUNK, D_HID), jnp.float32),
        pltpu.VMEM((CHUNK, D_HID), jnp.float32),
        pltpu.VMEM((ZROWS, D_HID), jnp.float32),
        pltpu.VMEM_SHARED((N_NODES, D_HID), jnp.float32),
        pltpu.VMEM_SHARED((N_NODES, D_HID), jnp.float32),
        pltpu.SemaphoreType.DMA,
    ],
)
def _sc_scatter(row, col, y, e, sums_out, cnts_out, *scratch):
    _sc_body(row, col, y, e, sums_out, cnts_out, *scratch)


def kernel(x, edge_index, edge_attr, u, batch, W1, b1, W2, b2):
    del u, batch
    row = edge_index[0].astype(jnp.int32)
    col = edge_index[1].astype(jnp.int32)
    W1a, W1b = W1[:D_FEAT], W1[D_FEAT:]
    W2a, W2b = W2[:D_FEAT], W2[D_FEAT:]
    b1r = b1.reshape(1, D_HID)
    b2r = b2.reshape(1, D_HID)

    y = pl.pallas_call(
        _y_body,
        out_shape=jax.ShapeDtypeStruct((N_NODES, D_HID), jnp.float32),
    )(x, W1a, b1r)

    eblk = 16000
    e = pl.pallas_call(
        _e_body,
        grid=(N_EDGES // eblk,),
        in_specs=[
            pl.BlockSpec((eblk, D_HID), lambda i: (i, 0)),
            pl.BlockSpec((D_HID, D_HID), lambda i: (0, 0)),
        ],
        out_specs=pl.BlockSpec((eblk, D_HID), lambda i: (i, 0)),
        out_shape=jax.ShapeDtypeStruct((N_EDGES, D_HID), jnp.float32),
    )(edge_attr, W1b)

    sums, cnts = _sc_scatter(row, col, y, e)

    out = pl.pallas_call(
        _fin_body,
        out_shape=jax.ShapeDtypeStruct((N_NODES, D_HID), jnp.float32),
    )(x, sums, cnts, W2a, W2b, b2r)
    return out


# speedup vs baseline: 5.3507x; 1.6392x over previous
import functools

import jax
import jax.numpy as jnp
from jax import lax
from jax.experimental import pallas as pl
from jax.experimental.pallas import tpu as pltpu
from jax.experimental.pallas import tpu_sc as plsc

N_NODES = 10000
N_EDGES = 320000
D_FEAT = 128
D_HID = 16

NUM_CORES = 2
NUM_SUBCORES = 16
NW = NUM_CORES * NUM_SUBCORES
EDGES_PER_W = N_EDGES // NW
CHUNK = 80
STEPS = EDGES_PER_W // CHUNK
ZROWS = 1000
SLAB = 624
TAIL = N_NODES - NUM_SUBCORES * SLAB


def _y_body(x_ref, w_ref, b_ref, o_ref):
    o_ref[...] = (
        jnp.dot(x_ref[...], w_ref[...], preferred_element_type=jnp.float32)
        + b_ref[...]
    )


def _e_body(ea_ref, w_ref, o_ref):
    o_ref[...] = jnp.dot(ea_ref[...], w_ref[...],
                         preferred_element_type=jnp.float32)


def _fin_body(x_ref, s_ref, c_ref, wa_ref, wb_ref, b_ref, o_ref):
    sums = s_ref[0] + s_ref[1]
    cnts = jnp.maximum(c_ref[0] + c_ref[1], 1.0)
    agg = sums / cnts
    o_ref[...] = jnp.maximum(
        jnp.dot(x_ref[...], wa_ref[...], preferred_element_type=jnp.float32)
        + jnp.dot(agg, wb_ref[...], preferred_element_type=jnp.float32)
        + b_ref[...],
        0.0,
    )


def _sc_body(row_ref, col_ref, y_ref, e_ref, sums_out, cnts_out,
             ridx_all, cidx_all, rows0, rows1, ev0, ev1, ones, zv,
             sums_sh, cnts_sh, sem_g0, sem_g1, sem_e0, sem_e1,
             sem_s, sem_c):
    c = lax.axis_index("c")
    s = lax.axis_index("s")
    wid = s * NUM_CORES + c
    rows = (rows0, rows1)
    ev = (ev0, ev1)
    sem_g = (sem_g0, sem_g1)
    sem_e = (sem_e0, sem_e1)

    pltpu.sync_copy(row_ref.at[wid], ridx_all)
    pltpu.sync_copy(col_ref.at[wid], cidx_all)

    def _fill(i, _):
        ones[i] = jnp.ones((16,), jnp.float32)
        return _
    lax.fori_loop(0, CHUNK, _fill, None)

    def _zfill(i, _):
        zv[i] = jnp.zeros((16,), jnp.float32)
        return _
    lax.fori_loop(0, ZROWS, _zfill, None)

    @pl.when(s == 0)
    def _zero_spmem():
        def _z(k, _):
            sl = pl.ds(k * ZROWS, ZROWS)
            pltpu.sync_copy(zv, sums_sh.at[sl])
            pltpu.sync_copy(zv, cnts_sh.at[sl])
            return _
        lax.fori_loop(0, N_NODES // ZROWS, _z, None)

    plsc.subcore_barrier()

    def _issue(j, b):
        pltpu.async_copy(y_ref.at[ridx_all.at[j]], rows[b], sem_g[b])
        base = wid * EDGES_PER_W + j * CHUNK
        pltpu.async_copy(e_ref.at[pl.ds(base, CHUNK)], ev[b], sem_e[b])

    def _process(j, b, start_next):
        nb = 1 - b
        if start_next:
            @pl.when(j + 1 < STEPS)
            def _pn():
                _issue(j + 1, nb)
        pltpu.make_async_copy(e_ref.at[pl.ds(0, CHUNK)], rows[b], sem_g[b]).wait()
        pltpu.make_async_copy(e_ref.at[pl.ds(0, CHUNK)], ev[b], sem_e[b]).wait()
        for i in range(CHUNK):
            rows[b][i] = jnp.maximum(rows[b][i] + ev[b][i], 0.0)
        h1 = pltpu.async_copy(rows[b], sums_sh.at[cidx_all.at[j]], sem_s,
                              add=True)
        h2 = pltpu.async_copy(ones, cnts_sh.at[cidx_all.at[j]], sem_c,
                              add=True)
        h1.wait()
        h2.wait()

    _issue(0, 0)
    _process(0, 0, True)

    def _pair(k, _):
        _process(2 * k + 1, 1, True)
        _process(2 * k + 2, 0, True)
        return _
    lax.fori_loop(0, (STEPS - 1) // 2, _pair, None)

    plsc.subcore_barrier()

    sl = pl.ds(s * SLAB, SLAB)
    pltpu.sync_copy(sums_sh.at[sl], sums_out.at[c, sl])
    pltpu.sync_copy(cnts_sh.at[sl], cnts_out.at[c, sl])

    @pl.when(s == 0)
    def _tail():
        tl = pl.ds(NUM_SUBCORES * SLAB, TAIL)
        pltpu.sync_copy(sums_sh.at[tl], sums_out.at[c, tl])
        pltpu.sync_copy(cnts_sh.at[tl], cnts_out.at[c, tl])


@functools.partial(
    pl.kernel,
    mesh=plsc.VectorSubcoreMesh(core_axis_name="c", subcore_axis_name="s"),
    compiler_params=pltpu.CompilerParams(use_tc_tiling_on_sc=False),
    out_type=[
        jax.ShapeDtypeStruct((NUM_CORES, N_NODES, D_HID), jnp.float32),
        jax.ShapeDtypeStruct((NUM_CORES, N_NODES, D_HID), jnp.float32),
    ],
    scratch_types=[
        pltpu.VMEM((STEPS, CHUNK), jnp.int32),
        pltpu.VMEM((STEPS, CHUNK), jnp.int32),
        pltpu.VMEM((CHUNK, D_HID), jnp.float32),
        pltpu.VMEM((CHUNK, D_HID), jnp.float32),
        pltpu.VMEM((CHUNK, D_HID), jnp.float32),
        pltpu.VMEM((CHUNK, D_HID), jnp.float32),
        pltpu.VMEM((CHUNK, D_HID), jnp.float32),
        pltpu.VMEM((ZROWS, D_HID), jnp.float32),
        pltpu.VMEM_SHARED((N_NODES, D_HID), jnp.float32),
        pltpu.VMEM_SHARED((N_NODES, D_HID), jnp.float32),
        pltpu.SemaphoreType.DMA,
        pltpu.SemaphoreType.DMA,
        pltpu.SemaphoreType.DMA,
        pltpu.SemaphoreType.DMA,
        pltpu.SemaphoreType.DMA,
        pltpu.SemaphoreType.DMA,
    ],
)
def _sc_scatter(row, col, y, e, sums_out, cnts_out, *scratch):
    _sc_body(row, col, y, e, sums_out, cnts_out, *scratch)


def kernel(x, edge_index, edge_attr, u, batch, W1, b1, W2, b2):
    del u, batch
    row = edge_index[0].astype(jnp.int32).reshape(NW, STEPS, CHUNK)
    col = edge_index[1].astype(jnp.int32).reshape(NW, STEPS, CHUNK)
    W1a, W1b = W1[:D_FEAT], W1[D_FEAT:]
    W2a, W2b = W2[:D_FEAT], W2[D_FEAT:]
    b1r = b1.reshape(1, D_HID)
    b2r = b2.reshape(1, D_HID)

    y = pl.pallas_call(
        _y_body,
        out_shape=jax.ShapeDtypeStruct((N_NODES, D_HID), jnp.float32),
    )(x, W1a, b1r)

    eblk = 16000
    e = pl.pallas_call(
        _e_body,
        grid=(N_EDGES // eblk,),
        in_specs=[
            pl.BlockSpec((eblk, D_HID), lambda i: (i, 0)),
            pl.BlockSpec((D_HID, D_HID), lambda i: (0, 0)),
        ],
        out_specs=pl.BlockSpec((eblk, D_HID), lambda i: (i, 0)),
        out_shape=jax.ShapeDtypeStruct((N_EDGES, D_HID), jnp.float32),
    )(edge_attr, W1b)

    sums, cnts = _sc_scatter(row, col, y, e)

    out = pl.pallas_call(
        _fin_body,
        out_shape=jax.ShapeDtypeStruct((N_NODES, D_HID), jnp.float32),
    )(x, sums, cnts, W2a, W2b, b2r)
    return out


# speedup vs baseline: 5.4597x; 1.0204x over previous
import functools

import jax
import jax.numpy as jnp
from jax import lax
from jax.experimental import pallas as pl
from jax.experimental.pallas import tpu as pltpu
from jax.experimental.pallas import tpu_sc as plsc

N_NODES = 10000
N_EDGES = 320000
D_FEAT = 128
D_HID = 16

NUM_CORES = 2
NUM_SUBCORES = 16
NW = NUM_CORES * NUM_SUBCORES
EDGES_PER_W = N_EDGES // NW
CHUNK = 80
STEPS = EDGES_PER_W // CHUNK
SLAB = 624
TAIL = N_NODES - NUM_SUBCORES * SLAB


def _y_body(x_ref, w_ref, b_ref, o_ref):
    o_ref[...] = (
        jnp.dot(x_ref[...], w_ref[...], preferred_element_type=jnp.float32)
        + b_ref[...]
    )


def _e_body(ea_ref, w_ref, o_ref):
    o_ref[...] = jnp.dot(ea_ref[...], w_ref[...],
                         preferred_element_type=jnp.float32)


def _fin_body(x_ref, s_ref, c_ref, wa_ref, wb_ref, b_ref, o_ref):
    sums = s_ref[0] + s_ref[1]
    cnts = jnp.maximum(c_ref[0] + c_ref[1], 1.0)
    agg = sums / cnts
    o_ref[...] = jnp.maximum(
        jnp.dot(x_ref[...], wa_ref[...], preferred_element_type=jnp.float32)
        + jnp.dot(agg, wb_ref[...], preferred_element_type=jnp.float32)
        + b_ref[...],
        0.0,
    )


def _sc_body(row_ref, col_ref, y_ref, e_ref, sums_out, cnts_out,
             ridx_all, cidx_all, rows0, rows1, ev0, ev1, ones, zv, czv,
             sums_sh, cnts_sh, sem_g0, sem_g1, sem_e0, sem_e1,
             sem_s, sem_c):
    c = lax.axis_index("c")
    s = lax.axis_index("s")
    wid = s * NUM_CORES + c
    rows = (rows0, rows1)
    ev = (ev0, ev1)
    sem_g = (sem_g0, sem_g1)
    sem_e = (sem_e0, sem_e1)

    pltpu.sync_copy(row_ref.at[wid], ridx_all)
    pltpu.sync_copy(col_ref.at[wid], cidx_all)

    def _fill(i, _):
        ones[pl.ds(i * 16, 16)] = jnp.ones((16,), jnp.float32)
        return _
    lax.fori_loop(0, CHUNK // 16, _fill, None)

    def _zfill(i, _):
        zv[i] = jnp.zeros((16,), jnp.float32)
        return _
    lax.fori_loop(0, SLAB, _zfill, None)

    def _czfill(i, _):
        czv[pl.ds(i * 16, 16)] = jnp.zeros((16,), jnp.float32)
        return _
    lax.fori_loop(0, SLAB // 16, _czfill, None)

    sl = pl.ds(s * SLAB, SLAB)
    pltpu.sync_copy(zv, sums_sh.at[sl])
    pltpu.sync_copy(czv, cnts_sh.at[sl])

    @pl.when(s == 0)
    def _zero_tail():
        tl = pl.ds(NUM_SUBCORES * SLAB, TAIL)
        pltpu.sync_copy(zv.at[pl.ds(0, TAIL)], sums_sh.at[tl])
        pltpu.sync_copy(czv.at[pl.ds(0, TAIL)], cnts_sh.at[tl])

    plsc.subcore_barrier()

    def _issue(j, b):
        pltpu.async_copy(y_ref.at[ridx_all.at[j]], rows[b], sem_g[b])
        base = wid * EDGES_PER_W + j * CHUNK
        pltpu.async_copy(e_ref.at[pl.ds(base, CHUNK)], ev[b], sem_e[b])

    def _process(j, b, start_next):
        nb = 1 - b
        if start_next:
            @pl.when(j + 1 < STEPS)
            def _pn():
                _issue(j + 1, nb)
        pltpu.make_async_copy(e_ref.at[pl.ds(0, CHUNK)], rows[b], sem_g[b]).wait()
        pltpu.make_async_copy(e_ref.at[pl.ds(0, CHUNK)], ev[b], sem_e[b]).wait()
        for i in range(CHUNK):
            rows[b][i] = jnp.maximum(rows[b][i] + ev[b][i], 0.0)
        h1 = pltpu.async_copy(rows[b], sums_sh.at[cidx_all.at[j]], sem_s,
                              add=True)
        h2 = pltpu.async_copy(ones, cnts_sh.at[cidx_all.at[j]], sem_c,
                              add=True)
        h1.wait()
        h2.wait()

    _issue(0, 0)
    _process(0, 0, True)

    def _pair(k, _):
        _process(2 * k + 1, 1, True)
        _process(2 * k + 2, 0, True)
        return _
    lax.fori_loop(0, (STEPS - 1) // 2, _pair, None)

    plsc.subcore_barrier()

    sl2 = pl.ds(s * SLAB, SLAB)
    pltpu.sync_copy(sums_sh.at[sl2], sums_out.at[c, sl2])
    pltpu.sync_copy(cnts_sh.at[sl2], cnts_out.at[c, sl2])

    @pl.when(s == 0)
    def _tail():
        tl = pl.ds(NUM_SUBCORES * SLAB, TAIL)
        pltpu.sync_copy(sums_sh.at[tl], sums_out.at[c, tl])
        pltpu.sync_copy(cnts_sh.at[tl], cnts_out.at[c, tl])


@functools.partial(
    pl.kernel,
    mesh=plsc.VectorSubcoreMesh(core_axis_name="c", subcore_axis_name="s"),
    compiler_params=pltpu.CompilerParams(use_tc_tiling_on_sc=False),
    out_type=[
        jax.ShapeDtypeStruct((NUM_CORES, N_NODES, D_HID), jnp.float32),
        jax.ShapeDtypeStruct((NUM_CORES, N_NODES), jnp.float32),
    ],
    scratch_types=[
        pltpu.VMEM((STEPS, CHUNK), jnp.int32),
        pltpu.VMEM((STEPS, CHUNK), jnp.int32),
        pltpu.VMEM((CHUNK, D_HID), jnp.float32),
        pltpu.VMEM((CHUNK, D_HID), jnp.float32),
        pltpu.VMEM((CHUNK, D_HID), jnp.float32),
        pltpu.VMEM((CHUNK, D_HID), jnp.float32),
        pltpu.VMEM((CHUNK,), jnp.float32),
        pltpu.VMEM((SLAB, D_HID), jnp.float32),
        pltpu.VMEM((SLAB,), jnp.float32),
        pltpu.VMEM_SHARED((N_NODES, D_HID), jnp.float32),
        pltpu.VMEM_SHARED((N_NODES,), jnp.float32),
        pltpu.SemaphoreType.DMA,
        pltpu.SemaphoreType.DMA,
        pltpu.SemaphoreType.DMA,
        pltpu.SemaphoreType.DMA,
        pltpu.SemaphoreType.DMA,
        pltpu.SemaphoreType.DMA,
    ],
)
def _sc_scatter(row, col, y, e, sums_out, cnts_out, *scratch):
    _sc_body(row, col, y, e, sums_out, cnts_out, *scratch)


def kernel(x, edge_index, edge_attr, u, batch, W1, b1, W2, b2):
    del u, batch
    row = edge_index[0].astype(jnp.int32).reshape(NW, STEPS, CHUNK)
    col = edge_index[1].astype(jnp.int32).reshape(NW, STEPS, CHUNK)
    W1a, W1b = W1[:D_FEAT], W1[D_FEAT:]
    W2a, W2b = W2[:D_FEAT], W2[D_FEAT:]
    b1r = b1.reshape(1, D_HID)
    b2r = b2.reshape(1, D_HID)

    y = pl.pallas_call(
        _y_body,
        out_shape=jax.ShapeDtypeStruct((N_NODES, D_HID), jnp.float32),
    )(x, W1a, b1r)

    eblk = 16000
    e = pl.pallas_call(
        _e_body,
        grid=(N_EDGES // eblk,),
        in_specs=[
            pl.BlockSpec((eblk, D_HID), lambda i: (i, 0)),
            pl.BlockSpec((D_HID, D_HID), lambda i: (0, 0)),
        ],
        out_specs=pl.BlockSpec((eblk, D_HID), lambda i: (i, 0)),
        out_shape=jax.ShapeDtypeStruct((N_EDGES, D_HID), jnp.float32),
    )(edge_attr, W1b)

    sums, cnts = _sc_scatter(row, col, y, e)
    cnts = cnts.reshape(NUM_CORES, N_NODES, 1)

    out = pl.pallas_call(
        _fin_body,
        out_shape=jax.ShapeDtypeStruct((N_NODES, D_HID), jnp.float32),
    )(x, sums, cnts, W2a, W2b, b2r)
    return out


# speedup vs baseline: 5.5304x; 1.0130x over previous
import functools

import jax
import jax.numpy as jnp
from jax import lax
from jax.experimental import pallas as pl
from jax.experimental.pallas import tpu as pltpu
from jax.experimental.pallas import tpu_sc as plsc

N_NODES = 10000
N_EDGES = 320000
D_FEAT = 128
D_HID = 16

NUM_CORES = 2
NUM_SUBCORES = 16
NW = NUM_CORES * NUM_SUBCORES
EDGES_PER_W = N_EDGES // NW
CHUNK = 80
STEPS = EDGES_PER_W // CHUNK
SLAB = 624
TAIL = N_NODES - NUM_SUBCORES * SLAB


def _y_body(x_ref, w_ref, b_ref, o_ref):
    o_ref[...] = (
        jnp.dot(x_ref[...], w_ref[...], preferred_element_type=jnp.float32)
        + b_ref[...]
    )


def _e_body(ea_ref, w_ref, o_ref):
    o_ref[...] = jnp.dot(ea_ref[...], w_ref[...],
                         preferred_element_type=jnp.float32)


def _fin_body(x_ref, s_ref, c_ref, wa_ref, wb_ref, b_ref, o_ref):
    sums = s_ref[0] + s_ref[1]
    cnts = jnp.maximum(c_ref[0] + c_ref[1], 1.0)
    agg = sums / cnts
    o_ref[...] = jnp.maximum(
        jnp.dot(x_ref[...], wa_ref[...], preferred_element_type=jnp.float32)
        + jnp.dot(agg, wb_ref[...], preferred_element_type=jnp.float32)
        + b_ref[...],
        0.0,
    )


def _sc_body(row_ref, col_ref, y_ref, e_ref, sums_out, cnts_out,
             ridx_all, cidx_all, rows0, rows1, ev0, ev1, ones, zv,
             sums_sh, cnts_sh, sem_g0, sem_g1, sem_e0, sem_e1,
             sem_s, sem_c):
    c = lax.axis_index("c")
    s = lax.axis_index("s")
    wid = s * NUM_CORES + c
    rows = (rows0, rows1)
    ev = (ev0, ev1)
    sem_g = (sem_g0, sem_g1)
    sem_e = (sem_e0, sem_e1)

    pltpu.sync_copy(row_ref.at[wid], ridx_all)
    pltpu.sync_copy(col_ref.at[wid], cidx_all)

    def _fill(i, _):
        ones[i] = jnp.ones((16,), jnp.float32)
        return _
    lax.fori_loop(0, CHUNK, _fill, None)

    def _zfill(i, _):
        zv[i] = jnp.zeros((16,), jnp.float32)
        return _
    lax.fori_loop(0, SLAB, _zfill, None)

    sl = pl.ds(s * SLAB, SLAB)
    pltpu.sync_copy(zv, sums_sh.at[sl])
    pltpu.sync_copy(zv, cnts_sh.at[sl])

    @pl.when(s == 0)
    def _zero_tail():
        tl = pl.ds(NUM_SUBCORES * SLAB, TAIL)
        pltpu.sync_copy(zv.at[pl.ds(0, TAIL)], sums_sh.at[tl])
        pltpu.sync_copy(zv.at[pl.ds(0, TAIL)], cnts_sh.at[tl])

    plsc.subcore_barrier()

    def _issue(j, b):
        pltpu.async_copy(y_ref.at[ridx_all.at[j]], rows[b], sem_g[b])
        base = wid * EDGES_PER_W + j * CHUNK
        pltpu.async_copy(e_ref.at[pl.ds(base, CHUNK)], ev[b], sem_e[b])

    def _process(j, b, start_next):
        nb = 1 - b
        if start_next:
            @pl.when(j + 1 < STEPS)
            def _pn():
                _issue(j + 1, nb)
        pltpu.make_async_copy(e_ref.at[pl.ds(0, CHUNK)], rows[b], sem_g[b]).wait()
        pltpu.make_async_copy(e_ref.at[pl.ds(0, CHUNK)], ev[b], sem_e[b]).wait()
        for i in range(CHUNK):
            rows[b][i] = jnp.maximum(rows[b][i] + ev[b][i], 0.0)
        h1 = pltpu.async_copy(rows[b], sums_sh.at[cidx_all.at[j]], sem_s,
                              add=True)
        h2 = pltpu.async_copy(ones, cnts_sh.at[cidx_all.at[j]], sem_c,
                              add=True)
        h1.wait()
        h2.wait()

    _issue(0, 0)
    _process(0, 0, True)

    def _pair(k, _):
        _process(2 * k + 1, 1, True)
        _process(2 * k + 2, 0, True)
        return _
    lax.fori_loop(0, (STEPS - 1) // 2, _pair, None)

    plsc.subcore_barrier()

    sl2 = pl.ds(s * SLAB, SLAB)
    pltpu.sync_copy(sums_sh.at[sl2], sums_out.at[c, sl2])
    pltpu.sync_copy(cnts_sh.at[sl2], cnts_out.at[c, sl2])

    @pl.when(s == 0)
    def _tail():
        tl = pl.ds(NUM_SUBCORES * SLAB, TAIL)
        pltpu.sync_copy(sums_sh.at[tl], sums_out.at[c, tl])
        pltpu.sync_copy(cnts_sh.at[tl], cnts_out.at[c, tl])


@functools.partial(
    pl.kernel,
    mesh=plsc.VectorSubcoreMesh(core_axis_name="c", subcore_axis_name="s"),
    compiler_params=pltpu.CompilerParams(use_tc_tiling_on_sc=False),
    out_type=[
        jax.ShapeDtypeStruct((NUM_CORES, N_NODES, D_HID), jnp.float32),
        jax.ShapeDtypeStruct((NUM_CORES, N_NODES, D_HID), jnp.float32),
    ],
    scratch_types=[
        pltpu.VMEM((STEPS, CHUNK), jnp.int32),
        pltpu.VMEM((STEPS, CHUNK), jnp.int32),
        pltpu.VMEM((CHUNK, D_HID), jnp.float32),
        pltpu.VMEM((CHUNK, D_HID), jnp.float32),
        pltpu.VMEM((CHUNK, D_HID), jnp.float32),
        pltpu.VMEM((CHUNK, D_HID), jnp.float32),
        pltpu.VMEM((CHUNK, D_HID), jnp.float32),
        pltpu.VMEM((SLAB, D_HID), jnp.float32),
        pltpu.VMEM_SHARED((N_NODES, D_HID), jnp.float32),
        pltpu.VMEM_SHARED((N_NODES, D_HID), jnp.float32),
        pltpu.SemaphoreType.DMA,
        pltpu.SemaphoreType.DMA,
        pltpu.SemaphoreType.DMA,
        pltpu.SemaphoreType.DMA,
        pltpu.SemaphoreType.DMA,
        pltpu.SemaphoreType.DMA,
    ],
)
def _sc_scatter(row, col, y, e, sums_out, cnts_out, *scratch):
    _sc_body(row, col, y, e, sums_out, cnts_out, *scratch)


def kernel(x, edge_index, edge_attr, u, batch, W1, b1, W2, b2):
    del u, batch
    row = edge_index[0].astype(jnp.int32).reshape(NW, STEPS, CHUNK)
    col = edge_index[1].astype(jnp.int32).reshape(NW, STEPS, CHUNK)
    W1a, W1b = W1[:D_FEAT], W1[D_FEAT:]
    W2a, W2b = W2[:D_FEAT], W2[D_FEAT:]
    b1r = b1.reshape(1, D_HID)
    b2r = b2.reshape(1, D_HID)

    y = pl.pallas_call(
        _y_body,
        out_shape=jax.ShapeDtypeStruct((N_NODES, D_HID), jnp.float32),
    )(x, W1a, b1r)

    eblk = 16000
    e = pl.pallas_call(
        _e_body,
        grid=(N_EDGES // eblk,),
        in_specs=[
            pl.BlockSpec((eblk, D_HID), lambda i: (i, 0)),
            pl.BlockSpec((D_HID, D_HID), lambda i: (0, 0)),
        ],
        out_specs=pl.BlockSpec((eblk, D_HID), lambda i: (i, 0)),
        out_shape=jax.ShapeDtypeStruct((N_EDGES, D_HID), jnp.float32),
    )(edge_attr, W1b)

    sums, cnts = _sc_scatter(row, col, y, e)

    out = pl.pallas_call(
        _fin_body,
        out_shape=jax.ShapeDtypeStruct((N_NODES, D_HID), jnp.float32),
    )(x, sums, cnts, W2a, W2b, b2r)
    return out


# speedup vs baseline: 5.6268x; 1.0174x over previous
import functools

import jax
import jax.numpy as jnp
from jax import lax
from jax.experimental import pallas as pl
from jax.experimental.pallas import tpu as pltpu
from jax.experimental.pallas import tpu_sc as plsc

N_NODES = 10000
N_EDGES = 320000
D_FEAT = 128
D_HID = 16

NUM_CORES = 2
NUM_SUBCORES = 16
NW = NUM_CORES * NUM_SUBCORES
EDGES_PER_W = N_EDGES // NW
CHUNK = 80
STEPS = EDGES_PER_W // CHUNK
SLAB = 624
TAIL = N_NODES - NUM_SUBCORES * SLAB


def _y_body(x_ref, w_ref, b_ref, o_ref):
    o_ref[...] = (
        jnp.dot(x_ref[...], w_ref[...], preferred_element_type=jnp.float32)
        + b_ref[...]
    )


def _e_body(ea_ref, w_ref, o_ref):
    o_ref[...] = jnp.dot(ea_ref[...], w_ref[...],
                         preferred_element_type=jnp.float32)


def _fin_body(x_ref, s_ref, c_ref, wa_ref, wb_ref, b_ref, o_ref):
    sums = s_ref[0] + s_ref[1]
    cnts = jnp.maximum(c_ref[0] + c_ref[1], 1.0)
    agg = sums / cnts
    o_ref[...] = jnp.maximum(
        jnp.dot(x_ref[...], wa_ref[...], preferred_element_type=jnp.float32)
        + jnp.dot(agg, wb_ref[...], preferred_element_type=jnp.float32)
        + b_ref[...],
        0.0,
    )


def _sc_body(row_ref, col_ref, y_ref, e_ref, sums_out, cnts_out,
             ridx_all, cidx_all, rows0, rows1, rows2, ev0, ev1, ev2,
             ones, zv, sums_sh, cnts_sh,
             sem_g0, sem_g1, sem_g2, sem_e0, sem_e1, sem_e2,
             sem_s0, sem_s1, sem_s2, sem_c0, sem_c1, sem_c2):
    c = lax.axis_index("c")
    s = lax.axis_index("s")
    wid = s * NUM_CORES + c
    rows = (rows0, rows1, rows2)
    ev = (ev0, ev1, ev2)
    sem_g = (sem_g0, sem_g1, sem_g2)
    sem_e = (sem_e0, sem_e1, sem_e2)
    sem_s = (sem_s0, sem_s1, sem_s2)
    sem_c = (sem_c0, sem_c1, sem_c2)

    pltpu.sync_copy(row_ref.at[wid], ridx_all)
    pltpu.sync_copy(col_ref.at[wid], cidx_all)

    def _fill(i, _):
        ones[i] = jnp.ones((16,), jnp.float32)
        return _
    lax.fori_loop(0, CHUNK, _fill, None)

    def _zfill(i, _):
        zv[i] = jnp.zeros((16,), jnp.float32)
        return _
    lax.fori_loop(0, SLAB, _zfill, None)

    sl = pl.ds(s * SLAB, SLAB)
    pltpu.sync_copy(zv, sums_sh.at[sl])
    pltpu.sync_copy(zv, cnts_sh.at[sl])

    @pl.when(s == 0)
    def _zero_tail():
        tl = pl.ds(NUM_SUBCORES * SLAB, TAIL)
        pltpu.sync_copy(zv.at[pl.ds(0, TAIL)], sums_sh.at[tl])
        pltpu.sync_copy(zv.at[pl.ds(0, TAIL)], cnts_sh.at[tl])

    plsc.subcore_barrier()

    def _issue(j, b):
        pltpu.async_copy(y_ref.at[ridx_all.at[j]], rows[b], sem_g[b])
        base = wid * EDGES_PER_W + j * CHUNK
        pltpu.async_copy(e_ref.at[pl.ds(base, CHUNK)], ev[b], sem_e[b])

    def _drain_scatter(b):
        pltpu.make_async_copy(e_ref.at[pl.ds(0, CHUNK)], rows[b], sem_s[b]).wait()
        pltpu.make_async_copy(e_ref.at[pl.ds(0, CHUNK)], ones, sem_c[b]).wait()

    def _process(j, b):
        nb = (b + 1) % 3

        @pl.when(j >= 2)
        def _ds():
            _drain_scatter(nb)

        @pl.when(j + 1 < STEPS)
        def _pn():
            _issue(j + 1, nb)

        pltpu.make_async_copy(e_ref.at[pl.ds(0, CHUNK)], rows[b], sem_g[b]).wait()
        pltpu.make_async_copy(e_ref.at[pl.ds(0, CHUNK)], ev[b], sem_e[b]).wait()
        for i in range(CHUNK):
            rows[b][i] = jnp.maximum(rows[b][i] + ev[b][i], 0.0)
        pltpu.async_copy(rows[b], sums_sh.at[cidx_all.at[j]], sem_s[b],
                         add=True)
        pltpu.async_copy(ones, cnts_sh.at[cidx_all.at[j]], sem_c[b],
                         add=True)

    _issue(0, 0)
    _process(0, 0)
    _process(1, 1)

    def _trip(k, _):
        _process(3 * k + 2, 2)
        _process(3 * k + 3, 0)
        _process(3 * k + 4, 1)
        return _
    lax.fori_loop(0, (STEPS - 2) // 3, _trip, None)

    _drain_scatter(0)
    _drain_scatter(1)

    plsc.subcore_barrier()

    sl2 = pl.ds(s * SLAB, SLAB)
    pltpu.sync_copy(sums_sh.at[sl2], sums_out.at[c, sl2])
    pltpu.sync_copy(cnts_sh.at[sl2], cnts_out.at[c, sl2])

    @pl.when(s == 0)
    def _tail():
        tl = pl.ds(NUM_SUBCORES * SLAB, TAIL)
        pltpu.sync_copy(sums_sh.at[tl], sums_out.at[c, tl])
        pltpu.sync_copy(cnts_sh.at[tl], cnts_out.at[c, tl])


@functools.partial(
    pl.kernel,
    mesh=plsc.VectorSubcoreMesh(core_axis_name="c", subcore_axis_name="s"),
    compiler_params=pltpu.CompilerParams(use_tc_tiling_on_sc=False),
    out_type=[
        jax.ShapeDtypeStruct((NUM_CORES, N_NODES, D_HID), jnp.float32),
        jax.ShapeDtypeStruct((NUM_CORES, N_NODES, D_HID), jnp.float32),
    ],
    scratch_types=[
        pltpu.VMEM((STEPS, CHUNK), jnp.int32),
        pltpu.VMEM((STEPS, CHUNK), jnp.int32),
        pltpu.VMEM((CHUNK, D_HID), jnp.float32),
        pltpu.VMEM((CHUNK, D_HID), jnp.float32),
        pltpu.VMEM((CHUNK, D_HID), jnp.float32),
        pltpu.VMEM((CHUNK, D_HID), jnp.float32),
        pltpu.VMEM((CHUNK, D_HID), jnp.float32),
        pltpu.VMEM((CHUNK, D_HID), jnp.float32),
        pltpu.VMEM((CHUNK, D_HID), jnp.float32),
        pltpu.VMEM((SLAB, D_HID), jnp.float32),
        pltpu.VMEM_SHARED((N_NODES, D_HID), jnp.float32),
        pltpu.VMEM_SHARED((N_NODES, D_HID), jnp.float32),
    ] + [pltpu.SemaphoreType.DMA] * 12,
)
def _sc_scatter(row, col, y, e, sums_out, cnts_out, *scratch):
    _sc_body(row, col, y, e, sums_out, cnts_out, *scratch)


def kernel(x, edge_index, edge_attr, u, batch, W1, b1, W2, b2):
    del u, batch
    row = edge_index[0].astype(jnp.int32).reshape(NW, STEPS, CHUNK)
    col = edge_index[1].astype(jnp.int32).reshape(NW, STEPS, CHUNK)
    W1a, W1b = W1[:D_FEAT], W1[D_FEAT:]
    W2a, W2b = W2[:D_FEAT], W2[D_FEAT:]
    b1r = b1.reshape(1, D_HID)
    b2r = b2.reshape(1, D_HID)

    y = pl.pallas_call(
        _y_body,
        out_shape=jax.ShapeDtypeStruct((N_NODES, D_HID), jnp.float32),
    )(x, W1a, b1r)

    eblk = 16000
    e = pl.pallas_call(
        _e_body,
        grid=(N_EDGES // eblk,),
        in_specs=[
            pl.BlockSpec((eblk, D_HID), lambda i: (i, 0)),
            pl.BlockSpec((D_HID, D_HID), lambda i: (0, 0)),
        ],
        out_specs=pl.BlockSpec((eblk, D_HID), lambda i: (i, 0)),
        out_shape=jax.ShapeDtypeStruct((N_EDGES, D_HID), jnp.float32),
    )(edge_attr, W1b)

    sums, cnts = _sc_scatter(row, col, y, e)

    out = pl.pallas_call(
        _fin_body,
        out_shape=jax.ShapeDtypeStruct((N_NODES, D_HID), jnp.float32),
    )(x, sums, cnts, W2a, W2b, b2r)
    return out


# speedup vs baseline: 5.6385x; 1.0021x over previous
import functools

import jax
import jax.numpy as jnp
from jax import lax
from jax.experimental import pallas as pl
from jax.experimental.pallas import tpu as pltpu
from jax.experimental.pallas import tpu_sc as plsc

N_NODES = 10000
N_EDGES = 320000
D_FEAT = 128
D_HID = 16

NUM_CORES = 2
NUM_SUBCORES = 16
NW = NUM_CORES * NUM_SUBCORES
EDGES_PER_W = N_EDGES // NW
CHUNK = 80
STEPS = EDGES_PER_W // CHUNK
SLAB = 624
TAIL = N_NODES - NUM_SUBCORES * SLAB


def _y_body(x_ref, w_ref, b_ref, o_ref):
    o_ref[...] = (
        jnp.dot(x_ref[...], w_ref[...], preferred_element_type=jnp.float32)
        + b_ref[...]
    )


def _e_body(ea_ref, w_ref, o_ref):
    o_ref[...] = jnp.dot(ea_ref[...], w_ref[...],
                         preferred_element_type=jnp.float32)


def _fin_body(x_ref, s_ref, c_ref, wa_ref, wb_ref, b_ref, o_ref):
    sums = s_ref[0] + s_ref[1]
    cnts = jnp.maximum(c_ref[0] + c_ref[1], 1.0)
    agg = sums / cnts
    o_ref[...] = jnp.maximum(
        jnp.dot(x_ref[...], wa_ref[...], preferred_element_type=jnp.float32)
        + jnp.dot(agg, wb_ref[...], preferred_element_type=jnp.float32)
        + b_ref[...],
        0.0,
    )


def _sc_body(row_ref, col_ref, y_ref, e_ref, sums_out, cnts_out,
             ridx_all, cidx_all, rows0, rows1, rows2, ev0, ev1, ev2,
             zb0, zb1, zb2, ones, zv, sums_sh, cnts_sh,
             sem_g0, sem_g1, sem_g2, sem_e0, sem_e1, sem_e2,
             sem_s0, sem_s1, sem_s2, sem_c0, sem_c1, sem_c2):
    c = lax.axis_index("c")
    s = lax.axis_index("s")
    wid = s * NUM_CORES + c
    rows = (rows0, rows1, rows2)
    ev = (ev0, ev1, ev2)
    zb = (zb0, zb1, zb2)
    sem_g = (sem_g0, sem_g1, sem_g2)
    sem_e = (sem_e0, sem_e1, sem_e2)
    sem_s = (sem_s0, sem_s1, sem_s2)
    sem_c = (sem_c0, sem_c1, sem_c2)

    pltpu.sync_copy(row_ref.at[wid], ridx_all)
    pltpu.sync_copy(col_ref.at[wid], cidx_all)

    def _fill(i, _):
        ones[i] = jnp.ones((16,), jnp.float32)
        return _
    lax.fori_loop(0, CHUNK, _fill, None)

    def _zfill(i, _):
        zv[i] = jnp.zeros((16,), jnp.float32)
        return _
    lax.fori_loop(0, SLAB, _zfill, None)

    sl = pl.ds(s * SLAB, SLAB)
    pltpu.sync_copy(zv, sums_sh.at[sl])
    pltpu.sync_copy(zv, cnts_sh.at[sl])

    @pl.when(s == 0)
    def _zero_tail():
        tl = pl.ds(NUM_SUBCORES * SLAB, TAIL)
        pltpu.sync_copy(zv.at[pl.ds(0, TAIL)], sums_sh.at[tl])
        pltpu.sync_copy(zv.at[pl.ds(0, TAIL)], cnts_sh.at[tl])

    plsc.subcore_barrier()

    def _issue(j, b):
        pltpu.async_copy(y_ref.at[ridx_all.at[j]], rows[b], sem_g[b])
        base = wid * EDGES_PER_W + j * CHUNK
        pltpu.async_copy(e_ref.at[pl.ds(base, CHUNK)], ev[b], sem_e[b])

    def _drain_scatter(b):
        pltpu.make_async_copy(e_ref.at[pl.ds(0, CHUNK)], zb[b], sem_s[b]).wait()
        pltpu.make_async_copy(e_ref.at[pl.ds(0, CHUNK)], ones, sem_c[b]).wait()

    def _process(j, b):
        nb = (b + 1) % 3

        @pl.when(j >= 2)
        def _ds():
            _drain_scatter(nb)

        @pl.when(j + 1 < STEPS)
        def _pn():
            _issue(j + 1, nb)

        pltpu.async_copy(ones, cnts_sh.at[cidx_all.at[j]], sem_c[b],
                         add=True)
        pltpu.make_async_copy(e_ref.at[pl.ds(0, CHUNK)], rows[b], sem_g[b]).wait()
        pltpu.make_async_copy(e_ref.at[pl.ds(0, CHUNK)], ev[b], sem_e[b]).wait()
        for i in range(CHUNK):
            zb[b][i] = jnp.maximum(rows[b][i] + ev[b][i], 0.0)
        pltpu.async_copy(zb[b], sums_sh.at[cidx_all.at[j]], sem_s[b],
                         add=True)

    _issue(0, 0)
    _process(0, 0)
    _process(1, 1)

    def _trip(k, _):
        _process(3 * k + 2, 2)
        _process(3 * k + 3, 0)
        _process(3 * k + 4, 1)
        return _
    lax.fori_loop(0, (STEPS - 2) // 3, _trip, None)

    _drain_scatter(0)
    _drain_scatter(1)

    plsc.subcore_barrier()

    sl2 = pl.ds(s * SLAB, SLAB)
    pltpu.sync_copy(sums_sh.at[sl2], sums_out.at[c, sl2])
    pltpu.sync_copy(cnts_sh.at[sl2], cnts_out.at[c, sl2])

    @pl.when(s == 0)
    def _tail():
        tl = pl.ds(NUM_SUBCORES * SLAB, TAIL)
        pltpu.sync_copy(sums_sh.at[tl], sums_out.at[c, tl])
        pltpu.sync_copy(cnts_sh.at[tl], cnts_out.at[c, tl])


@functools.partial(
    pl.kernel,
    mesh=plsc.VectorSubcoreMesh(core_axis_name="c", subcore_axis_name="s"),
    compiler_params=pltpu.CompilerParams(use_tc_tiling_on_sc=False),
    out_type=[
        jax.ShapeDtypeStruct((NUM_CORES, N_NODES, D_HID), jnp.float32),
        jax.ShapeDtypeStruct((NUM_CORES, N_NODES, D_HID), jnp.float32),
    ],
    scratch_types=[
        pltpu.VMEM((STEPS, CHUNK), jnp.int32),
        pltpu.VMEM((STEPS, CHUNK), jnp.int32),
        pltpu.VMEM((CHUNK, D_HID), jnp.float32),
        pltpu.VMEM((CHUNK, D_HID), jnp.float32),
        pltpu.VMEM((CHUNK, D_HID), jnp.float32),
        pltpu.VMEM((CHUNK, D_HID), jnp.float32),
        pltpu.VMEM((CHUNK, D_HID), jnp.float32),
        pltpu.VMEM((CHUNK, D_HID), jnp.float32),
        pltpu.VMEM((CHUNK, D_HID), jnp.float32),
        pltpu.VMEM((CHUNK, D_HID), jnp.float32),
        pltpu.VMEM((CHUNK, D_HID), jnp.float32),
        pltpu.VMEM((CHUNK, D_HID), jnp.float32),
        pltpu.VMEM((SLAB, D_HID), jnp.float32),
        pltpu.VMEM_SHARED((N_NODES, D_HID), jnp.float32),
        pltpu.VMEM_SHARED((N_NODES, D_HID), jnp.float32),
    ] + [pltpu.SemaphoreType.DMA] * 12,
)
def _sc_scatter(row, col, y, e, sums_out, cnts_out, *scratch):
    _sc_body(row, col, y, e, sums_out, cnts_out, *scratch)


def kernel(x, edge_index, edge_attr, u, batch, W1, b1, W2, b2):
    del u, batch
    row = edge_index[0].astype(jnp.int32).reshape(NW, STEPS, CHUNK)
    col = edge_index[1].astype(jnp.int32).reshape(NW, STEPS, CHUNK)
    W1a, W1b = W1[:D_FEAT], W1[D_FEAT:]
    W2a, W2b = W2[:D_FEAT], W2[D_FEAT:]
    b1r = b1.reshape(1, D_HID)
    b2r = b2.reshape(1, D_HID)

    y = pl.pallas_call(
        _y_body,
        out_shape=jax.ShapeDtypeStruct((N_NODES, D_HID), jnp.float32),
    )(x, W1a, b1r)

    eblk = 16000
    e = pl.pallas_call(
        _e_body,
        grid=(N_EDGES // eblk,),
        in_specs=[
            pl.BlockSpec((eblk, D_HID), lambda i: (i, 0)),
            pl.BlockSpec((D_HID, D_HID), lambda i: (0, 0)),
        ],
        out_specs=pl.BlockSpec((eblk, D_HID), lambda i: (i, 0)),
        out_shape=jax.ShapeDtypeStruct((N_EDGES, D_HID), jnp.float32),
    )(edge_attr, W1b)

    sums, cnts = _sc_scatter(row, col, y, e)

    out = pl.pallas_call(
        _fin_body,
        out_shape=jax.ShapeDtypeStruct((N_NODES, D_HID), jnp.float32),
    )(x, sums, cnts, W2a, W2b, b2r)
    return out


# speedup vs baseline: 5.8494x; 1.0374x over previous
import functools

import jax
import jax.numpy as jnp
from jax import lax
from jax.experimental import pallas as pl
from jax.experimental.pallas import tpu as pltpu
from jax.experimental.pallas import tpu_sc as plsc

N_NODES = 10000
N_EDGES = 320000
D_FEAT = 128
D_HID = 16

NUM_CORES = 2
NUM_SUBCORES = 16
NW = NUM_CORES * NUM_SUBCORES
EDGES_PER_W = N_EDGES // NW
CHUNK = 80
STEPS = EDGES_PER_W // CHUNK
SLAB = 624
TAIL = N_NODES - NUM_SUBCORES * SLAB


def _y_body(x_ref, w_ref, b_ref, o_ref):
    o_ref[...] = (
        jnp.dot(x_ref[...], w_ref[...], preferred_element_type=jnp.float32)
        + b_ref[...]
    )


def _e_body(ea_ref, w_ref, o_ref):
    o_ref[...] = jnp.dot(ea_ref[...], w_ref[...],
                         preferred_element_type=jnp.float32)


def _fin_body(x_ref, s_ref, c_ref, wa_ref, wb_ref, b_ref, o_ref):
    sums = s_ref[0] + s_ref[1]
    cnts = jnp.maximum(c_ref[0] + c_ref[1], 1.0)
    agg = sums / cnts
    o_ref[...] = jnp.maximum(
        jnp.dot(x_ref[...], wa_ref[...], preferred_element_type=jnp.float32)
        + jnp.dot(agg, wb_ref[...], preferred_element_type=jnp.float32)
        + b_ref[...],
        0.0,
    )


def _sc_body(row_ref, col_ref, y_ref, e_ref, sums_out, cnts_out,
             ridx_all, cidx_all, rows0, rows1, rows2, ev0, ev1, ev2,
             zb0, zb1, zb2, ones, zv, sums_sh, cnts_sh,
             sem_g0, sem_g1, sem_g2, sem_e0, sem_e1, sem_e2,
             sem_s0, sem_s1, sem_s2, sem_c0, sem_c1, sem_c2):
    c = lax.axis_index("c")
    s = lax.axis_index("s")
    wid = s * NUM_CORES + c
    rows = (rows0, rows1, rows2)
    ev = (ev0, ev1, ev2)
    zb = (zb0, zb1, zb2)
    sem_g = (sem_g0, sem_g1, sem_g2)
    sem_e = (sem_e0, sem_e1, sem_e2)
    sem_s = (sem_s0, sem_s1, sem_s2)
    sem_c = (sem_c0, sem_c1, sem_c2)

    pltpu.sync_copy(row_ref.at[wid], ridx_all)
    pltpu.sync_copy(col_ref.at[wid], cidx_all)

    def _fill(i, _):
        ones[i] = jnp.ones((16,), jnp.float32)
        return _
    lax.fori_loop(0, CHUNK, _fill, None)

    def _zfill(i, _):
        zv[i] = jnp.zeros((16,), jnp.float32)
        return _
    lax.fori_loop(0, SLAB, _zfill, None)

    sl = pl.ds(s * SLAB, SLAB)
    pltpu.sync_copy(zv, sums_sh.at[sl])
    pltpu.sync_copy(zv, cnts_sh.at[sl])

    @pl.when(s == 0)
    def _zero_tail():
        tl = pl.ds(NUM_SUBCORES * SLAB, TAIL)
        pltpu.sync_copy(zv.at[pl.ds(0, TAIL)], sums_sh.at[tl])
        pltpu.sync_copy(zv.at[pl.ds(0, TAIL)], cnts_sh.at[tl])

    plsc.subcore_barrier()

    def _issue(j, b):
        pltpu.async_copy(y_ref.at[ridx_all.at[j]], rows[b], sem_g[b])
        base = wid * EDGES_PER_W + j * CHUNK
        pltpu.async_copy(e_ref.at[pl.ds(base, CHUNK)], ev[b], sem_e[b])

    def _drain_scatter(b):
        pltpu.make_async_copy(e_ref.at[pl.ds(0, CHUNK)], zb[b], sem_s[b]).wait()
        pltpu.make_async_copy(e_ref.at[pl.ds(0, CHUNK)], ones, sem_c[b]).wait()

    def _process(j, b):
        @pl.when(j + 2 < STEPS)
        def _pn():
            _issue(j + 2, (b + 2) % 3)

        @pl.when(j >= 3)
        def _ds():
            _drain_scatter(b)

        pltpu.async_copy(ones, cnts_sh.at[cidx_all.at[j]], sem_c[b],
                         add=True)
        pltpu.make_async_copy(e_ref.at[pl.ds(0, CHUNK)], rows[b], sem_g[b]).wait()
        pltpu.make_async_copy(e_ref.at[pl.ds(0, CHUNK)], ev[b], sem_e[b]).wait()
        for i in range(CHUNK):
            zb[b][i] = jnp.maximum(rows[b][i] + ev[b][i], 0.0)
        pltpu.async_copy(zb[b], sums_sh.at[cidx_all.at[j]], sem_s[b],
                         add=True)

    _issue(0, 0)
    _issue(1, 1)
    _process(0, 0)
    _process(1, 1)

    def _trip(k, _):
        _process(3 * k + 2, 2)
        _process(3 * k + 3, 0)
        _process(3 * k + 4, 1)
        return _
    lax.fori_loop(0, (STEPS - 2) // 3, _trip, None)

    _drain_scatter(2)
    _drain_scatter(0)
    _drain_scatter(1)

    plsc.subcore_barrier()

    sl2 = pl.ds(s * SLAB, SLAB)
    pltpu.sync_copy(sums_sh.at[sl2], sums_out.at[c, sl2])
    pltpu.sync_copy(cnts_sh.at[sl2], cnts_out.at[c, sl2])

    @pl.when(s == 0)
    def _tail():
        tl = pl.ds(NUM_SUBCORES * SLAB, TAIL)
        pltpu.sync_copy(sums_sh.at[tl], sums_out.at[c, tl])
        pltpu.sync_copy(cnts_sh.at[tl], cnts_out.at[c, tl])


@functools.partial(
    pl.kernel,
    mesh=plsc.VectorSubcoreMesh(core_axis_name="c", subcore_axis_name="s"),
    compiler_params=pltpu.CompilerParams(use_tc_tiling_on_sc=False),
    out_type=[
        jax.ShapeDtypeStruct((NUM_CORES, N_NODES, D_HID), jnp.float32),
        jax.ShapeDtypeStruct((NUM_CORES, N_NODES, D_HID), jnp.float32),
    ],
    scratch_types=[
        pltpu.VMEM((STEPS, CHUNK), jnp.int32),
        pltpu.VMEM((STEPS, CHUNK), jnp.int32),
        pltpu.VMEM((CHUNK, D_HID), jnp.float32),
        pltpu.VMEM((CHUNK, D_HID), jnp.float32),
        pltpu.VMEM((CHUNK, D_HID), jnp.float32),
        pltpu.VMEM((CHUNK, D_HID), jnp.float32),
        pltpu.VMEM((CHUNK, D_HID), jnp.float32),
        pltpu.VMEM((CHUNK, D_HID), jnp.float32),
        pltpu.VMEM((CHUNK, D_HID), jnp.float32),
        pltpu.VMEM((CHUNK, D_HID), jnp.float32),
        pltpu.VMEM((CHUNK, D_HID), jnp.float32),
        pltpu.VMEM((CHUNK, D_HID), jnp.float32),
        pltpu.VMEM((SLAB, D_HID), jnp.float32),
        pltpu.VMEM_SHARED((N_NODES, D_HID), jnp.float32),
        pltpu.VMEM_SHARED((N_NODES, D_HID), jnp.float32),
    ] + [pltpu.SemaphoreType.DMA] * 12,
)
def _sc_scatter(row, col, y, e, sums_out, cnts_out, *scratch):
    _sc_body(row, col, y, e, sums_out, cnts_out, *scratch)


def kernel(x, edge_index, edge_attr, u, batch, W1, b1, W2, b2):
    del u, batch
    row = edge_index[0].astype(jnp.int32).reshape(NW, STEPS, CHUNK)
    col = edge_index[1].astype(jnp.int32).reshape(NW, STEPS, CHUNK)
    W1a, W1b = W1[:D_FEAT], W1[D_FEAT:]
    W2a, W2b = W2[:D_FEAT], W2[D_FEAT:]
    b1r = b1.reshape(1, D_HID)
    b2r = b2.reshape(1, D_HID)

    y = pl.pallas_call(
        _y_body,
        out_shape=jax.ShapeDtypeStruct((N_NODES, D_HID), jnp.float32),
    )(x, W1a, b1r)

    eblk = 16000
    e = pl.pallas_call(
        _e_body,
        grid=(N_EDGES // eblk,),
        in_specs=[
            pl.BlockSpec((eblk, D_HID), lambda i: (i, 0)),
            pl.BlockSpec((D_HID, D_HID), lambda i: (0, 0)),
        ],
        out_specs=pl.BlockSpec((eblk, D_HID), lambda i: (i, 0)),
        out_shape=jax.ShapeDtypeStruct((N_EDGES, D_HID), jnp.float32),
    )(edge_attr, W1b)

    sums, cnts = _sc_scatter(row, col, y, e)

    out = pl.pallas_call(
        _fin_body,
        out_shape=jax.ShapeDtypeStruct((N_NODES, D_HID), jnp.float32),
    )(x, sums, cnts, W2a, W2b, b2r)
    return out


# speedup vs baseline: 5.9146x; 1.0112x over previous
import functools

import jax
import jax.numpy as jnp
from jax import lax
from jax.experimental import pallas as pl
from jax.experimental.pallas import tpu as pltpu
from jax.experimental.pallas import tpu_sc as plsc

N_NODES = 10000
N_EDGES = 320000
D_FEAT = 128
D_HID = 16

NUM_CORES = 2
NUM_SUBCORES = 16
NW = NUM_CORES * NUM_SUBCORES
EDGES_PER_W = N_EDGES // NW
CHUNK = 80
STEPS = EDGES_PER_W // CHUNK
SLAB = 624
TAIL = N_NODES - NUM_SUBCORES * SLAB


def _y_body(x_ref, w_ref, b_ref, o_ref):
    o_ref[...] = (
        jnp.dot(x_ref[...], w_ref[...], preferred_element_type=jnp.float32)
        + b_ref[...]
    )


def _e_body(ea_ref, w_ref, o_ref):
    o_ref[...] = jnp.dot(ea_ref[...], w_ref[...],
                         preferred_element_type=jnp.float32)


def _fin_body(x_ref, s_ref, c_ref, wa_ref, wb_ref, b_ref, o_ref):
    sums = s_ref[0] + s_ref[1]
    cnts = jnp.maximum(c_ref[0] + c_ref[1], 1.0)
    agg = sums / cnts
    o_ref[...] = jnp.maximum(
        jnp.dot(x_ref[...], wa_ref[...], preferred_element_type=jnp.float32)
        + jnp.dot(agg, wb_ref[...], preferred_element_type=jnp.float32)
        + b_ref[...],
        0.0,
    )


def _sc_body(row_ref, col_ref, y_ref, e_ref, sums_out, cnts_out,
             ridx_all, cidx_all, rows0, rows1, rows2, ev0, ev1, ev2,
             zb0, zb1, zb2, ones, zv, sums_sh, cnts_sh, y_sh,
             sem_g0, sem_g1, sem_g2, sem_e0, sem_e1, sem_e2,
             sem_s0, sem_s1, sem_s2, sem_c0, sem_c1, sem_c2):
    c = lax.axis_index("c")
    s = lax.axis_index("s")
    wid = s * NUM_CORES + c
    rows = (rows0, rows1, rows2)
    ev = (ev0, ev1, ev2)
    zb = (zb0, zb1, zb2)
    sem_g = (sem_g0, sem_g1, sem_g2)
    sem_e = (sem_e0, sem_e1, sem_e2)
    sem_s = (sem_s0, sem_s1, sem_s2)
    sem_c = (sem_c0, sem_c1, sem_c2)

    pltpu.sync_copy(row_ref.at[wid], ridx_all)
    pltpu.sync_copy(col_ref.at[wid], cidx_all)

    def _fill(i, _):
        ones[i] = jnp.ones((16,), jnp.float32)
        return _
    lax.fori_loop(0, CHUNK, _fill, None)

    def _zfill(i, _):
        zv[i] = jnp.zeros((16,), jnp.float32)
        return _
    lax.fori_loop(0, SLAB, _zfill, None)

    sl = pl.ds(s * SLAB, SLAB)
    pltpu.sync_copy(zv, sums_sh.at[sl])
    pltpu.sync_copy(zv, cnts_sh.at[sl])
    pltpu.sync_copy(y_ref.at[sl], y_sh.at[sl])

    @pl.when(s == 0)
    def _zero_tail():
        tl = pl.ds(NUM_SUBCORES * SLAB, TAIL)
        pltpu.sync_copy(zv.at[pl.ds(0, TAIL)], sums_sh.at[tl])
        pltpu.sync_copy(zv.at[pl.ds(0, TAIL)], cnts_sh.at[tl])
        pltpu.sync_copy(y_ref.at[tl], y_sh.at[tl])

    plsc.subcore_barrier()

    def _issue(j, b):
        pltpu.async_copy(y_sh.at[ridx_all.at[j]], rows[b], sem_g[b])
        base = wid * EDGES_PER_W + j * CHUNK
        pltpu.async_copy(e_ref.at[pl.ds(base, CHUNK)], ev[b], sem_e[b])

    def _drain_scatter(b):
        pltpu.make_async_copy(e_ref.at[pl.ds(0, CHUNK)], zb[b], sem_s[b]).wait()
        pltpu.make_async_copy(e_ref.at[pl.ds(0, CHUNK)], ones, sem_c[b]).wait()

    def _process(j, b):
        @pl.when(j + 2 < STEPS)
        def _pn():
            _issue(j + 2, (b + 2) % 3)

        @pl.when(j >= 3)
        def _ds():
            _drain_scatter(b)

        pltpu.async_copy(ones, cnts_sh.at[cidx_all.at[j]], sem_c[b],
                         add=True)
        pltpu.make_async_copy(e_ref.at[pl.ds(0, CHUNK)], rows[b], sem_g[b]).wait()
        pltpu.make_async_copy(e_ref.at[pl.ds(0, CHUNK)], ev[b], sem_e[b]).wait()
        for i in range(CHUNK):
            zb[b][i] = jnp.maximum(rows[b][i] + ev[b][i], 0.0)
        pltpu.async_copy(zb[b], sums_sh.at[cidx_all.at[j]], sem_s[b],
                         add=True)

    _issue(0, 0)
    _issue(1, 1)
    _process(0, 0)
    _process(1, 1)

    def _trip(k, _):
        _process(3 * k + 2, 2)
        _process(3 * k + 3, 0)
        _process(3 * k + 4, 1)
        return _
    lax.fori_loop(0, (STEPS - 2) // 3, _trip, None)

    _drain_scatter(2)
    _drain_scatter(0)
    _drain_scatter(1)

    plsc.subcore_barrier()

    sl2 = pl.ds(s * SLAB, SLAB)
    pltpu.sync_copy(sums_sh.at[sl2], sums_out.at[c, sl2])
    pltpu.sync_copy(cnts_sh.at[sl2], cnts_out.at[c, sl2])

    @pl.when(s == 0)
    def _tail():
        tl = pl.ds(NUM_SUBCORES * SLAB, TAIL)
        pltpu.sync_copy(sums_sh.at[tl], sums_out.at[c, tl])
        pltpu.sync_copy(cnts_sh.at[tl], cnts_out.at[c, tl])


@functools.partial(
    pl.kernel,
    mesh=plsc.VectorSubcoreMesh(core_axis_name="c", subcore_axis_name="s"),
    compiler_params=pltpu.CompilerParams(use_tc_tiling_on_sc=False),
    out_type=[
        jax.ShapeDtypeStruct((NUM_CORES, N_NODES, D_HID), jnp.float32),
        jax.ShapeDtypeStruct((NUM_CORES, N_NODES, D_HID), jnp.float32),
    ],
    scratch_types=[
        pltpu.VMEM((STEPS, CHUNK), jnp.int32),
        pltpu.VMEM((STEPS, CHUNK), jnp.int32),
        pltpu.VMEM((CHUNK, D_HID), jnp.float32),
        pltpu.VMEM((CHUNK, D_HID), jnp.float32),
        pltpu.VMEM((CHUNK, D_HID), jnp.float32),
        pltpu.VMEM((CHUNK, D_HID), jnp.float32),
        pltpu.VMEM((CHUNK, D_HID), jnp.float32),
        pltpu.VMEM((CHUNK, D_HID), jnp.float32),
        pltpu.VMEM((CHUNK, D_HID), jnp.float32),
        pltpu.VMEM((CHUNK, D_HID), jnp.float32),
        pltpu.VMEM((CHUNK, D_HID), jnp.float32),
        pltpu.VMEM((CHUNK, D_HID), jnp.float32),
        pltpu.VMEM((SLAB, D_HID), jnp.float32),
        pltpu.VMEM_SHARED((N_NODES, D_HID), jnp.float32),
        pltpu.VMEM_SHARED((N_NODES, D_HID), jnp.float32),
        pltpu.VMEM_SHARED((N_NODES, D_HID), jnp.float32),
    ] + [pltpu.SemaphoreType.DMA] * 12,
)
def _sc_scatter(row, col, y, e, sums_out, cnts_out, *scratch):
    _sc_body(row, col, y, e, sums_out, cnts_out, *scratch)


def kernel(x, edge_index, edge_attr, u, batch, W1, b1, W2, b2):
    del u, batch
    row = edge_index[0].astype(jnp.int32).reshape(NW, STEPS, CHUNK)
    col = edge_index[1].astype(jnp.int32).reshape(NW, STEPS, CHUNK)
    W1a, W1b = W1[:D_FEAT], W1[D_FEAT:]
    W2a, W2b = W2[:D_FEAT], W2[D_FEAT:]
    b1r = b1.reshape(1, D_HID)
    b2r = b2.reshape(1, D_HID)

    y = pl.pallas_call(
        _y_body,
        out_shape=jax.ShapeDtypeStruct((N_NODES, D_HID), jnp.float32),
    )(x, W1a, b1r)

    eblk = 16000
    e = pl.pallas_call(
        _e_body,
        grid=(N_EDGES // eblk,),
        in_specs=[
            pl.BlockSpec((eblk, D_HID), lambda i: (i, 0)),
            pl.BlockSpec((D_HID, D_HID), lambda i: (0, 0)),
        ],
        out_specs=pl.BlockSpec((eblk, D_HID), lambda i: (i, 0)),
        out_shape=jax.ShapeDtypeStruct((N_EDGES, D_HID), jnp.float32),
    )(edge_attr, W1b)

    sums, cnts = _sc_scatter(row, col, y, e)

    out = pl.pallas_call(
        _fin_body,
        out_shape=jax.ShapeDtypeStruct((N_NODES, D_HID), jnp.float32),
    )(x, sums, cnts, W2a, W2b, b2r)
    return out
